# Initial kernel scaffold; baseline (speedup 1.0000x reference)
import jax
import jax.numpy as jnp
from jax.experimental import pallas as pl


def kernel(user, item, ue_gmf, ie_gmf, ue_mlp, ie_mlp, W1, b1, W2, b2, W3, b3, Wo, bo):
    raise NotImplementedError("write your pallas kernel here")


# speedup vs baseline: 4.2873x; 4.2873x over previous
import functools
import jax
import jax.numpy as jnp
from jax import lax
from jax.experimental import pallas as pl
from jax.experimental.pallas import tpu as pltpu
from jax.experimental.pallas import tpu_sc as plsc

_B = 16384
_D = 128

_NC = 2
_NS = 16
_NW = _NC * _NS
_BPW = _B // _NW


def _sc_gather_body(user_hbm, item_hbm, ug_hbm, ig_hbm, um_hbm, im_hbm,
                    out_gu, out_gi, out_mu, out_mi,
                    uidx_v, iidx_v, rows_v, sem):
    wid = lax.axis_index("s") * _NC + lax.axis_index("c")
    base = wid * _BPW
    pltpu.sync_copy(user_hbm.at[pl.ds(base, _BPW)], uidx_v)
    pltpu.sync_copy(item_hbm.at[pl.ds(base, _BPW)], iidx_v)

    pltpu.async_copy(ug_hbm.at[uidx_v], rows_v, sem).wait()
    pltpu.sync_copy(rows_v, out_gu.at[pl.ds(base, _BPW)])

    pltpu.async_copy(ig_hbm.at[iidx_v], rows_v, sem).wait()
    pltpu.sync_copy(rows_v, out_gi.at[pl.ds(base, _BPW)])

    pltpu.async_copy(um_hbm.at[uidx_v], rows_v, sem).wait()
    pltpu.sync_copy(rows_v, out_mu.at[pl.ds(base, _BPW)])

    pltpu.async_copy(im_hbm.at[iidx_v], rows_v, sem).wait()
    pltpu.sync_copy(rows_v, out_mi.at[pl.ds(base, _BPW)])


@functools.cache
def _sc_gather():
    return pl.kernel(
        _sc_gather_body,
        out_type=[jax.ShapeDtypeStruct((_B, _D), jnp.float32)] * 4,
        mesh=plsc.VectorSubcoreMesh(core_axis_name="c", subcore_axis_name="s"),
        scratch_types=[
            pltpu.VMEM((_BPW,), jnp.int32),
            pltpu.VMEM((_BPW,), jnp.int32),
            pltpu.VMEM((_BPW, _D), jnp.float32),
            pltpu.SemaphoreType.DMA,
        ],
    )


_BLK = 1024


def _dense_body(gu_ref, gi_ref, mu_ref, mi_ref,
                w1a_ref, w1b_ref, b1_ref, w2_ref, b2_ref, w3_ref, b3_ref,
                wog_ref, woh_ref, bo_ref, out_ref):
    h = jnp.maximum(
        jnp.dot(mu_ref[...], w1a_ref[...], preferred_element_type=jnp.float32)
        + jnp.dot(mi_ref[...], w1b_ref[...], preferred_element_type=jnp.float32)
        + b1_ref[...], 0.0)
    h = jnp.maximum(
        jnp.dot(h, w2_ref[...], preferred_element_type=jnp.float32)
        + b2_ref[...], 0.0)
    h = jnp.maximum(
        jnp.dot(h, w3_ref[...], preferred_element_type=jnp.float32)
        + b3_ref[...], 0.0)
    gmf = gu_ref[...] * gi_ref[...]
    logits = (jnp.sum(gmf * wog_ref[...], axis=1)
              + jnp.sum(h * woh_ref[...], axis=1)
              + bo_ref[0, 0])
    out_ref[...] = logits


def _dense(gu, gi, mu, mi, W1, b1, W2, b2, W3, b3, Wo, bo):
    w1a = W1[:_D]
    w1b = W1[_D:]
    wog = Wo[:_D, 0].reshape(1, _D)
    woh = Wo[_D:, 0].reshape(1, -1)
    b1r = b1.reshape(1, -1)
    b2r = b2.reshape(1, -1)
    b3r = b3.reshape(1, -1)
    bor = bo.reshape(1, 1)
    grid = _B // _BLK
    blk_in = pl.BlockSpec((_BLK, _D), lambda i: (i, 0))
    rep = lambda shape: pl.BlockSpec(shape, lambda i: tuple(0 for _ in shape))
    return pl.pallas_call(
        _dense_body,
        grid=(grid,),
        in_specs=[blk_in, blk_in, blk_in, blk_in,
                  rep(w1a.shape), rep(w1b.shape), rep(b1r.shape),
                  rep(W2.shape), rep(b2r.shape),
                  rep(W3.shape), rep(b3r.shape),
                  rep(wog.shape), rep(woh.shape), rep(bor.shape)],
        out_specs=pl.BlockSpec((_BLK,), lambda i: (i,)),
        out_shape=jax.ShapeDtypeStruct((_B,), jnp.float32),
    )(gu, gi, mu, mi, w1a, w1b, b1r, W2, b2r, W3, b3r, wog, woh, bor)


@jax.jit
def kernel(user, item, ue_gmf, ie_gmf, ue_mlp, ie_mlp,
           W1, b1, W2, b2, W3, b3, Wo, bo):
    gu, gi, mu, mi = _sc_gather()(user, item, ue_gmf, ie_gmf, ue_mlp, ie_mlp)
    return _dense(gu, gi, mu, mi, W1, b1, W2, b2, W3, b3, Wo, bo)


# speedup vs baseline: 4.3206x; 1.0078x over previous
import functools
import jax
import jax.numpy as jnp
from jax import lax
from jax.experimental import pallas as pl
from jax.experimental.pallas import tpu as pltpu
from jax.experimental.pallas import tpu_sc as plsc

_B = 16384
_D = 128

_NC = 2
_NS = 16
_NW = _NC * _NS
_BPW = _B // _NW


_C = 128
_K = _BPW // _C
_NB = 4


def _sc_gather_body(user_hbm, item_hbm, ug_hbm, ig_hbm, um_hbm, im_hbm,
                    out_gu, out_gi, out_mu, out_mi,
                    uidx_v, iidx_v, *bufs_and_sems):
    bufs = bufs_and_sems[:_NB]
    gsem = bufs_and_sems[_NB:2 * _NB]
    wsem = bufs_and_sems[2 * _NB:3 * _NB]
    wid = lax.axis_index("s") * _NC + lax.axis_index("c")
    base = wid * _BPW
    pltpu.sync_copy(user_hbm.at[pl.ds(base, _BPW)], uidx_v)
    pltpu.sync_copy(item_hbm.at[pl.ds(base, _BPW)], iidx_v)

    tasks = []
    for k in range(_K):
        off = k * _C
        for tbl, idx, out in ((ug_hbm, uidx_v, out_gu),
                              (ig_hbm, iidx_v, out_gi),
                              (um_hbm, uidx_v, out_mu),
                              (im_hbm, iidx_v, out_mi)):
            tasks.append((tbl, idx, out, off))

    n = len(tasks)
    wb = [None] * _NB

    def fire(i):
        tbl, idx, out, off = tasks[i]
        s = i % _NB
        if wb[s] is not None:
            wb[s].wait()
        return pltpu.async_copy(tbl.at[idx.at[pl.ds(off, _C)]], bufs[s], gsem[s])

    g = fire(0)
    for i in range(n):
        s = i % _NB
        nxt = fire(i + 1) if i + 1 < n else None
        g.wait()
        _, _, out, off = tasks[i]
        wb[s] = pltpu.make_async_copy(bufs[s], out.at[pl.ds(base + off, _C)],
                                      wsem[s])
        wb[s].start()
        g = nxt
    for s in range(_NB):
        if wb[s] is not None:
            wb[s].wait()


@functools.cache
def _sc_gather():
    return pl.kernel(
        _sc_gather_body,
        out_type=[jax.ShapeDtypeStruct((_B, _D), jnp.float32)] * 4,
        mesh=plsc.VectorSubcoreMesh(core_axis_name="c", subcore_axis_name="s"),
        scratch_types=(
            [pltpu.VMEM((_BPW,), jnp.int32),
             pltpu.VMEM((_BPW,), jnp.int32)]
            + [pltpu.VMEM((_C, _D), jnp.float32)] * _NB
            + [pltpu.SemaphoreType.DMA] * (2 * _NB)
        ),
    )


_BLK = 1024


def _dense_body(gu_ref, gi_ref, mu_ref, mi_ref,
                w1a_ref, w1b_ref, b1_ref, w2_ref, b2_ref, w3_ref, b3_ref,
                wog_ref, woh_ref, bo_ref, out_ref):
    h = jnp.maximum(
        jnp.dot(mu_ref[...], w1a_ref[...], preferred_element_type=jnp.float32)
        + jnp.dot(mi_ref[...], w1b_ref[...], preferred_element_type=jnp.float32)
        + b1_ref[...], 0.0)
    h = jnp.maximum(
        jnp.dot(h, w2_ref[...], preferred_element_type=jnp.float32)
        + b2_ref[...], 0.0)
    h = jnp.maximum(
        jnp.dot(h, w3_ref[...], preferred_element_type=jnp.float32)
        + b3_ref[...], 0.0)
    gmf = gu_ref[...] * gi_ref[...]
    logits = (jnp.sum(gmf * wog_ref[...], axis=1)
              + jnp.sum(h * woh_ref[...], axis=1)
              + bo_ref[0, 0])
    out_ref[...] = logits


def _dense(gu, gi, mu, mi, W1, b1, W2, b2, W3, b3, Wo, bo):
    w1a = W1[:_D]
    w1b = W1[_D:]
    wog = Wo[:_D, 0].reshape(1, _D)
    woh = Wo[_D:, 0].reshape(1, -1)
    b1r = b1.reshape(1, -1)
    b2r = b2.reshape(1, -1)
    b3r = b3.reshape(1, -1)
    bor = bo.reshape(1, 1)
    grid = _B // _BLK
    blk_in = pl.BlockSpec((_BLK, _D), lambda i: (i, 0))
    rep = lambda shape: pl.BlockSpec(shape, lambda i: tuple(0 for _ in shape))
    return pl.pallas_call(
        _dense_body,
        grid=(grid,),
        in_specs=[blk_in, blk_in, blk_in, blk_in,
                  rep(w1a.shape), rep(w1b.shape), rep(b1r.shape),
                  rep(W2.shape), rep(b2r.shape),
                  rep(W3.shape), rep(b3r.shape),
                  rep(wog.shape), rep(woh.shape), rep(bor.shape)],
        out_specs=pl.BlockSpec((_BLK,), lambda i: (i,)),
        out_shape=jax.ShapeDtypeStruct((_B,), jnp.float32),
    )(gu, gi, mu, mi, w1a, w1b, b1r, W2, b2r, W3, b3r, wog, woh, bor)


@jax.jit
def kernel(user, item, ue_gmf, ie_gmf, ue_mlp, ie_mlp,
           W1, b1, W2, b2, W3, b3, Wo, bo):
    gu, gi, mu, mi = _sc_gather()(user, item, ue_gmf, ie_gmf, ue_mlp, ie_mlp)
    return _dense(gu, gi, mu, mi, W1, b1, W2, b2, W3, b3, Wo, bo)


# speedup vs baseline: 4.4848x; 1.0380x over previous
import functools
import jax
import jax.numpy as jnp
from jax import lax
from jax.experimental import pallas as pl
from jax.experimental.pallas import tpu as pltpu
from jax.experimental.pallas import tpu_sc as plsc

_B = 16384
_D = 128

_NC = 2
_NS = 16
_NW = _NC * _NS

_C = 128
_NB = 4

_NSPLIT = 2
_BS = _B // _NSPLIT


def _sc_gather_body(user_hbm, item_hbm, ug_hbm, ig_hbm, um_hbm, im_hbm,
                    out_gu, out_gi, out_mu, out_mi,
                    uidx_v, iidx_v, *bufs_and_sems):
    bpw = _BS // _NW
    nk = bpw // _C
    bufs = bufs_and_sems[:_NB]
    gsem = bufs_and_sems[_NB:2 * _NB]
    wsem = bufs_and_sems[2 * _NB:3 * _NB]
    wid = lax.axis_index("s") * _NC + lax.axis_index("c")
    base = wid * bpw
    pltpu.sync_copy(user_hbm.at[pl.ds(base, bpw)], uidx_v)
    pltpu.sync_copy(item_hbm.at[pl.ds(base, bpw)], iidx_v)

    tasks = []
    for k in range(nk):
        off = k * _C
        for tbl, idx, out in ((ug_hbm, uidx_v, out_gu),
                              (ig_hbm, iidx_v, out_gi),
                              (um_hbm, uidx_v, out_mu),
                              (im_hbm, iidx_v, out_mi)):
            tasks.append((tbl, idx, out, off))

    n = len(tasks)
    wb = [None] * _NB

    def fire(i):
        tbl, idx, out, off = tasks[i]
        s = i % _NB
        if wb[s] is not None:
            wb[s].wait()
        return pltpu.async_copy(tbl.at[idx.at[pl.ds(off, _C)]], bufs[s], gsem[s])

    g = fire(0)
    for i in range(n):
        s = i % _NB
        nxt = fire(i + 1) if i + 1 < n else None
        g.wait()
        _, _, out, off = tasks[i]
        wb[s] = pltpu.make_async_copy(bufs[s], out.at[pl.ds(base + off, _C)],
                                      wsem[s])
        wb[s].start()
        g = nxt
    for s in range(_NB):
        if wb[s] is not None:
            wb[s].wait()


@functools.cache
def _sc_gather():
    bpw = _BS // _NW
    return pl.kernel(
        _sc_gather_body,
        out_type=[jax.ShapeDtypeStruct((_BS, _D), jnp.float32)] * 4,
        mesh=plsc.VectorSubcoreMesh(core_axis_name="c", subcore_axis_name="s"),
        scratch_types=(
            [pltpu.VMEM((bpw,), jnp.int32),
             pltpu.VMEM((bpw,), jnp.int32)]
            + [pltpu.VMEM((_C, _D), jnp.float32)] * _NB
            + [pltpu.SemaphoreType.DMA] * (2 * _NB)
        ),
    )


_BLK = 1024


def _dense_body(gu_ref, gi_ref, mu_ref, mi_ref,
                w1a_ref, w1b_ref, b1_ref, w2_ref, b2_ref, w3_ref, b3_ref,
                wog_ref, woh_ref, bo_ref, out_ref):
    h = jnp.maximum(
        jnp.dot(mu_ref[...], w1a_ref[...], preferred_element_type=jnp.float32)
        + jnp.dot(mi_ref[...], w1b_ref[...], preferred_element_type=jnp.float32)
        + b1_ref[...], 0.0)
    h = jnp.maximum(
        jnp.dot(h, w2_ref[...], preferred_element_type=jnp.float32)
        + b2_ref[...], 0.0)
    h = jnp.maximum(
        jnp.dot(h, w3_ref[...], preferred_element_type=jnp.float32)
        + b3_ref[...], 0.0)
    gmf = gu_ref[...] * gi_ref[...]
    rnd = lambda x: x.astype(jnp.bfloat16).astype(jnp.float32)
    logits = (jnp.sum(rnd(gmf) * rnd(wog_ref[...]), axis=1)
              + jnp.sum(rnd(h) * rnd(woh_ref[...]), axis=1)
              + bo_ref[0, 0])
    out_ref[...] = logits


def _dense(gu, gi, mu, mi, w1a, w1b, b1r, W2, b2r, W3, b3r, wog, woh, bor):
    nrows = gu.shape[0]
    grid = nrows // _BLK
    blk_in = pl.BlockSpec((_BLK, _D), lambda i: (i, 0))
    rep = lambda shape: pl.BlockSpec(shape, lambda i: tuple(0 for _ in shape))
    return pl.pallas_call(
        _dense_body,
        grid=(grid,),
        in_specs=[blk_in, blk_in, blk_in, blk_in,
                  rep(w1a.shape), rep(w1b.shape), rep(b1r.shape),
                  rep(W2.shape), rep(b2r.shape),
                  rep(W3.shape), rep(b3r.shape),
                  rep(wog.shape), rep(woh.shape), rep(bor.shape)],
        out_specs=pl.BlockSpec((_BLK,), lambda i: (i,)),
        out_shape=jax.ShapeDtypeStruct((nrows,), jnp.float32),
    )(gu, gi, mu, mi, w1a, w1b, b1r, W2, b2r, W3, b3r, wog, woh, bor)


@jax.jit
def kernel(user, item, ue_gmf, ie_gmf, ue_mlp, ie_mlp,
           W1, b1, W2, b2, W3, b3, Wo, bo):
    w1a = W1[:_D]
    w1b = W1[_D:]
    wog = Wo[:_D, 0].reshape(1, _D)
    woh = Wo[_D:, 0].reshape(1, -1)
    b1r = b1.reshape(1, -1)
    b2r = b2.reshape(1, -1)
    b3r = b3.reshape(1, -1)
    bor = bo.reshape(1, 1)
    sc = _sc_gather()
    outs = []
    for s in range(_NSPLIT):
        lo = s * _BS
        gu, gi, mu, mi = sc(user[lo:lo + _BS], item[lo:lo + _BS],
                            ue_gmf, ie_gmf, ue_mlp, ie_mlp)
        outs.append(_dense(gu, gi, mu, mi, w1a, w1b, b1r, W2, b2r, W3, b3r,
                           wog, woh, bor))
    return jnp.concatenate(outs)


# speedup vs baseline: 4.7030x; 1.0486x over previous
import functools
import jax
import jax.numpy as jnp
from jax import lax
from jax.experimental import pallas as pl
from jax.experimental.pallas import tpu as pltpu
from jax.experimental.pallas import tpu_sc as plsc

_B = 16384
_D = 128

_NC = 2
_NS = 16
_NW = _NC * _NS

_C = 64
_NSPLIT = 2
_BS = _B // _NSPLIT
_BPW = _BS // _NW
_NK = _BPW // _C


def _product_chunk(a_ref, b_ref):
    def row(r, _):
        for c in range(_D // 16):
            sl = pl.ds(c * 16, 16)
            a_ref[r, sl] = a_ref[r, sl] * b_ref[r, sl]
        return 0
    lax.fori_loop(0, _C, row, 0)


def _sc_body(off, user_hbm, item_hbm, ug_hbm, ig_hbm, um_hbm, im_hbm,
             out_gmf, out_mu, out_mi,
             uidx_v, iidx_v, *rest):
    bufs = rest[:8]
    gsem = rest[8:16]
    wsem = rest[16:22]
    A = bufs[0:2]
    Bb = bufs[2:4]
    MU = bufs[4:6]
    MI = bufs[6:8]
    wA = wsem[0:2]
    wMU = wsem[2:4]
    wMI = wsem[4:6]

    wid = lax.axis_index("s") * _NC + lax.axis_index("c")
    base = wid * _BPW
    pltpu.sync_copy(user_hbm.at[pl.ds(off + base, _BPW)], uidx_v)
    pltpu.sync_copy(item_hbm.at[pl.ds(off + base, _BPW)], iidx_v)

    gh = [None] * _NK
    wb = {}

    def fire_gathers(k):
        s = k % 2
        for key in (("A", s), ("MU", s), ("MI", s)):
            if key in wb:
                wb.pop(key).wait()
        co = k * _C
        uidx = uidx_v.at[pl.ds(co, _C)]
        iidx = iidx_v.at[pl.ds(co, _C)]
        gh[k] = (
            pltpu.async_copy(ug_hbm.at[uidx], A[s], gsem[0 + s]),
            pltpu.async_copy(ig_hbm.at[iidx], Bb[s], gsem[2 + s]),
            pltpu.async_copy(um_hbm.at[uidx], MU[s], gsem[4 + s]),
            pltpu.async_copy(im_hbm.at[iidx], MI[s], gsem[6 + s]),
        )

    def drain_chunk(j):
        s = j % 2
        dst = pl.ds(off + base + j * _C, _C)
        ga, gb, gmu, gmi = gh[j]
        ga.wait()
        gb.wait()
        _product_chunk(A[s], Bb[s])
        wb[("A", s)] = pltpu.make_async_copy(A[s], out_gmf.at[dst], wA[s])
        wb[("A", s)].start()
        gmu.wait()
        wb[("MU", s)] = pltpu.make_async_copy(MU[s], out_mu.at[dst], wMU[s])
        wb[("MU", s)].start()
        gmi.wait()
        wb[("MI", s)] = pltpu.make_async_copy(MI[s], out_mi.at[dst], wMI[s])
        wb[("MI", s)].start()

    fire_gathers(0)
    for k in range(1, _NK + 1):
        if k < _NK:
            fire_gathers(k)
        drain_chunk(k - 1)
    for h in wb.values():
        h.wait()


@functools.cache
def _sc_gather(off):
    return pl.kernel(
        functools.partial(_sc_body, off),
        out_type=[jax.ShapeDtypeStruct((_B, _D), jnp.float32)] * 3,
        mesh=plsc.VectorSubcoreMesh(core_axis_name="c", subcore_axis_name="s"),
        scratch_types=(
            [pltpu.VMEM((_BPW,), jnp.int32),
             pltpu.VMEM((_BPW,), jnp.int32)]
            + [pltpu.VMEM((_C, _D), jnp.float32)] * 8
            + [pltpu.SemaphoreType.DMA] * 14
        ),
    )


_BLK = 1024


def _dense_body(gmf_ref, mu_ref, mi_ref,
                w1a_ref, w1b_ref, b1_ref, w2_ref, b2_ref, w3_ref, b3_ref,
                wog_ref, woh_ref, bo_ref, out_ref):
    h = jnp.maximum(
        jnp.dot(mu_ref[...], w1a_ref[...], preferred_element_type=jnp.float32)
        + jnp.dot(mi_ref[...], w1b_ref[...], preferred_element_type=jnp.float32)
        + b1_ref[...], 0.0)
    h = jnp.maximum(
        jnp.dot(h, w2_ref[...], preferred_element_type=jnp.float32)
        + b2_ref[...], 0.0)
    h = jnp.maximum(
        jnp.dot(h, w3_ref[...], preferred_element_type=jnp.float32)
        + b3_ref[...], 0.0)
    rnd = lambda x: x.astype(jnp.bfloat16).astype(jnp.float32)
    logits = (jnp.sum(rnd(gmf_ref[...]) * rnd(wog_ref[...]), axis=1)
              + jnp.sum(rnd(h) * rnd(woh_ref[...]), axis=1)
              + bo_ref[0, 0])
    out_ref[...] = logits


def _dense(off, gmf, mu, mi, w1a, w1b, b1r, W2, b2r, W3, b3r, wog, woh, bor):
    grid = _BS // _BLK
    ob = off // _BLK
    blk_in = pl.BlockSpec((_BLK, _D), lambda i: (i + ob, 0))
    rep = lambda shape: pl.BlockSpec(shape, lambda i: tuple(0 for _ in shape))
    return pl.pallas_call(
        _dense_body,
        grid=(grid,),
        in_specs=[blk_in, blk_in, blk_in,
                  rep(w1a.shape), rep(w1b.shape), rep(b1r.shape),
                  rep(W2.shape), rep(b2r.shape),
                  rep(W3.shape), rep(b3r.shape),
                  rep(wog.shape), rep(woh.shape), rep(bor.shape)],
        out_specs=pl.BlockSpec((_BLK,), lambda i: (i,)),
        out_shape=jax.ShapeDtypeStruct((_BS,), jnp.float32),
    )(gmf, mu, mi, w1a, w1b, b1r, W2, b2r, W3, b3r, wog, woh, bor)


@jax.jit
def kernel(user, item, ue_gmf, ie_gmf, ue_mlp, ie_mlp,
           W1, b1, W2, b2, W3, b3, Wo, bo):
    w1a = W1[:_D]
    w1b = W1[_D:]
    wog = Wo[:_D, 0].reshape(1, _D)
    woh = Wo[_D:, 0].reshape(1, -1)
    b1r = b1.reshape(1, -1)
    b2r = b2.reshape(1, -1)
    b3r = b3.reshape(1, -1)
    bor = bo.reshape(1, 1)
    outs = []
    for s in range(_NSPLIT):
        off = s * _BS
        gmf, mu, mi = _sc_gather(off)(user, item, ue_gmf, ie_gmf,
                                      ue_mlp, ie_mlp)
        outs.append(_dense(off, gmf, mu, mi, w1a, w1b, b1r, W2, b2r,
                           W3, b3r, wog, woh, bor))
    return jnp.concatenate(outs)
